# Initial kernel scaffold; baseline (speedup 1.0000x reference)
import jax
import jax.numpy as jnp
from jax.experimental import pallas as pl


def kernel(memory_table, last_update, unique_node_ids, unique_messages, timestamps, w_ih, w_hh, b_ih, b_hh, fc_w, fc_b, lin_w, lin_b):
    raise NotImplementedError("write your pallas kernel here")


# speedup vs baseline: 1.8677x; 1.8677x over previous
import functools

import jax
import jax.numpy as jnp
from jax import lax
from jax.experimental import pallas as pl
from jax.experimental.pallas import tpu as pltpu
from jax.experimental.pallas import tpu_sc as plsc

M_ROWS = 100000
D = 256
B_ROWS = 16384
NC, NS = 2, 16
NW = NC * NS
BPW = B_ROWS // NW
CH = 128
NCH = BPW // CH

_mesh = plsc.VectorSubcoreMesh(core_axis_name="c", subcore_axis_name="s")


@functools.partial(
    pl.kernel,
    mesh=_mesh,
    out_type=jax.ShapeDtypeStruct((B_ROWS, D), jnp.float32),
    scratch_types=[
        pltpu.VMEM((CH,), jnp.int32),
        pltpu.VMEM((CH, D), jnp.float32),
        pltpu.SemaphoreType.DMA,
    ],
)
def _sc_gather(table_hbm, ids_hbm, out_hbm, idx_v, rows_v, sem):
  wid = lax.axis_index("s") * NC + lax.axis_index("c")
  base = wid * BPW
  for ch in range(NCH):
    off = base + ch * CH
    pltpu.sync_copy(ids_hbm.at[pl.ds(off, CH)], idx_v)
    pltpu.async_copy(table_hbm.at[idx_v], rows_v, sem).wait()
    pltpu.sync_copy(rows_v, out_hbm.at[pl.ds(off, CH)])


BLK = 512


def _gru_block(x_ref, h_ref, wih_ref, whh_ref, bih_ref, bhh_ref,
               fcw_ref, fcb_ref, linw_ref, linb_ref, out_ref):
  x = x_ref[...]
  h = h_ref[...]
  gi = jnp.dot(x, wih_ref[...], preferred_element_type=jnp.float32) + bih_ref[...]
  gh = jnp.dot(h, whh_ref[...], preferred_element_type=jnp.float32) + bhh_ref[...]
  r = jax.nn.sigmoid(gi[:, 0:D] + gh[:, 0:D])
  z = jax.nn.sigmoid(gi[:, D:2 * D] + gh[:, D:2 * D])
  n = jnp.tanh(gi[:, 2 * D:3 * D] + r * gh[:, 2 * D:3 * D])
  hy = (1.0 - z) * n + z * h
  pred = jnp.dot(hy, fcw_ref[...], preferred_element_type=jnp.float32) + fcb_ref[...]
  out_ref[...] = jnp.dot(pred, linw_ref[...], preferred_element_type=jnp.float32) + linb_ref[...]


_gru = pl.pallas_call(
    _gru_block,
    grid=(B_ROWS // BLK,),
    in_specs=[
        pl.BlockSpec((BLK, D), lambda i: (i, 0)),
        pl.BlockSpec((BLK, D), lambda i: (i, 0)),
        pl.BlockSpec((D, 3 * D), lambda i: (0, 0)),
        pl.BlockSpec((D, 3 * D), lambda i: (0, 0)),
        pl.BlockSpec((1, 3 * D), lambda i: (0, 0)),
        pl.BlockSpec((1, 3 * D), lambda i: (0, 0)),
        pl.BlockSpec((D, 64), lambda i: (0, 0)),
        pl.BlockSpec((1, 64), lambda i: (0, 0)),
        pl.BlockSpec((64, D), lambda i: (0, 0)),
        pl.BlockSpec((1, D), lambda i: (0, 0)),
    ],
    out_specs=pl.BlockSpec((BLK, D), lambda i: (i, 0)),
    out_shape=jax.ShapeDtypeStruct((B_ROWS, D), jnp.float32),
)


@functools.partial(
    pl.kernel,
    mesh=_mesh,
    out_type=(),
    scratch_types=[
        pltpu.VMEM((CH,), jnp.int32),
        pltpu.VMEM((CH,), jnp.int32),
        pltpu.VMEM((CH, D), jnp.float32),
        pltpu.VMEM((CH,), jnp.float32),
        pltpu.SemaphoreType.DMA,
    ],
)
def _sc_scatter(newmem_hbm, ids_hbm, occ_hbm, tsw_hbm, table_ref, lu_ref,
                occ_v, ids_v, rows_v, ts_v, sem):
  wid = lax.axis_index("s") * NC + lax.axis_index("c")
  base = wid * BPW
  for ch in range(NCH):
    off = base + ch * CH
    pltpu.sync_copy(occ_hbm.at[pl.ds(off, CH)], occ_v)
    pltpu.sync_copy(ids_hbm.at[pl.ds(off, CH)], ids_v)
    pltpu.sync_copy(tsw_hbm.at[pl.ds(off, CH)], ts_v)
    pltpu.async_copy(newmem_hbm.at[occ_v], rows_v, sem).wait()
    pltpu.async_copy(rows_v, table_ref.at[ids_v], sem).wait()
    pltpu.async_copy(ts_v, lu_ref.at[ids_v], sem).wait()


def kernel(memory_table, last_update, unique_node_ids, unique_messages,
           timestamps, w_ih, w_hh, b_ih, b_hh, fc_w, fc_b, lin_w, lin_b):
  ids = unique_node_ids

  iota = jnp.arange(B_ROWS, dtype=jnp.int32)
  lastpos = jnp.zeros((M_ROWS,), jnp.int32).at[ids].max(iota)
  lastocc = lastpos[ids]
  tsw = jnp.take(timestamps, lastocc)

  h = _sc_gather(memory_table, ids)

  new_mem = _gru(
      unique_messages, h,
      w_ih.T, w_hh.T,
      b_ih.reshape(1, 3 * D), b_hh.reshape(1, 3 * D),
      fc_w.T, fc_b.reshape(1, 64),
      lin_w.T, lin_b.reshape(1, D),
  )

  table_ref = jax.new_ref(memory_table)
  lu_ref = jax.new_ref(last_update)
  _sc_scatter(new_mem, ids, lastocc, tsw, table_ref, lu_ref)
  return jax.freeze(table_ref), jax.freeze(lu_ref)
